# baseline (speedup 1.0000x reference)
import math

import jax
import jax.numpy as jnp
from jax import lax
from jax.experimental import pallas as pl
from jax.experimental.pallas import tpu as pltpu
from jax.experimental.pallas import tpu_sc as plsc

DIM = 64
VOCAB = 1000000
SCALE = math.sqrt(DIM)
LANES = 16
CHUNK = 128
PADC = CHUNK + 5

VBLK = 256
PADV = VBLK + 5
N_FULL = VOCAB // VBLK
TAIL = VOCAB - N_FULL * VBLK


def _make_prep(num_workers: int):
    mesh = plsc.VectorSubcoreMesh(core_axis_name="c", subcore_axis_name="s")
    steps = 122
    n_main = steps * num_workers
    main_rows = n_main * VBLK

    def body(wt_hbm, t64_hbm, lin_hbm, a0, a1, w0, w1, sa0, sa1, sw0, sw1):
        nc = mesh.num_cores
        wid = lax.axis_index("s") * nc + lax.axis_index("c")
        ab = (a0, a1)
        wb = (w0, w1)
        sa = (sa0, sa1)
        sw = (sw0, sw1)

        lane_iota = lax.iota(jnp.int32, LANES)
        rows_q = [lane_iota + q * LANES for q in range(DIM // LANES)]

        def start_in(v0, b, width):
            v0 = pl.multiple_of(v0, 128)
            pltpu.async_copy(
                wt_hbm.at[:, pl.ds(v0, width)],
                ab[b].at[:, pl.ds(0, width)], sa[b])

        def wait_in(b, width):
            pltpu.make_async_copy(
                wt_hbm.at[:, pl.ds(0, width)],
                ab[b].at[:, pl.ds(0, width)], sa[b]).wait()

        def start_out(v0, b, width):
            r0 = pl.multiple_of(v0 // 2, 32)
            pltpu.async_copy(
                wb[b].at[pl.ds(0, width // 2)],
                lin_hbm.at[pl.ds(r0, width // 2)], sw[b])

        def wait_out(b, width):
            pltpu.make_async_copy(
                wb[b].at[pl.ds(0, width // 2)],
                lin_hbm.at[pl.ds(0, width // 2)], sw[b]).wait()

        def transpose(b, width):
            src = ab[b]
            dst = wb[b]

            @pl.loop(0, width, unroll=4)
            def _(v):
                vp = v // 2
                half = (v - vp * 2) * DIM
                v_vec = lane_iota * 0 + v
                for q in range(DIM // LANES):
                    vals = plsc.load_gather(src, [rows_q[q], v_vec])
                    dst[vp, pl.ds(half + q * LANES, LANES)] = vals * SCALE

        def v_of(j):
            return (j * num_workers + wid) * VBLK

        def pipe_step(j, b, *, out_wait, prefetch):
            wait_in(b, VBLK)
            if out_wait:
                wait_out(b, VBLK)
            transpose(b, VBLK)
            if prefetch:
                start_in(v_of(j + 2), b, VBLK)
            start_out(v_of(j), b, VBLK)

        start_in(v_of(0), 0, VBLK)
        start_in(v_of(1), 1, VBLK)
        pipe_step(0, 0, out_wait=False, prefetch=True)
        pipe_step(1, 1, out_wait=False, prefetch=True)

        @pl.loop(1, steps // 2 - 1)
        def _(g):
            pipe_step(2 * g, 0, out_wait=True, prefetch=True)
            pipe_step(2 * g + 1, 1, out_wait=True, prefetch=True)

        pipe_step(steps - 2, 0, out_wait=True, prefetch=False)
        pipe_step(steps - 1, 1, out_wait=True, prefetch=False)
        wait_out(0, VBLK)
        wait_out(1, VBLK)

        n_tail128 = (VOCAB - main_rows) // 128

        @pl.when(wid < n_tail128)
        def _():
            v0 = main_rows + wid * 128
            start_in(v0, 0, 128)
            wait_in(0, 128)
            transpose(0, 128)
            start_out(v0, 0, 128)
            wait_out(0, 128)

        @pl.when(wid == n_tail128)
        def _():
            pltpu.sync_copy(t64_hbm, wb[0].at[pl.ds(0, TAIL // 2)])
            pltpu.async_copy(
                wb[0].at[pl.ds(0, TAIL // 2)],
                lin_hbm.at[pl.ds((VOCAB - TAIL) // 2, TAIL // 2)], sw[0])
            wait_out(0, TAIL)

    kern = pl.kernel(
        body,
        out_type=jax.ShapeDtypeStruct((VOCAB // 2, 2 * DIM), jnp.float32),
        mesh=mesh,
        compiler_params=pltpu.CompilerParams(
            use_tc_tiling_on_sc=True, needs_layout_passes=False),
        scratch_types=[
            pltpu.VMEM((DIM, PADV), jnp.float32),
            pltpu.VMEM((DIM, PADV), jnp.float32),
            pltpu.VMEM((VBLK // 2, 2 * DIM), jnp.float32),
            pltpu.VMEM((VBLK // 2, 2 * DIM), jnp.float32),
            pltpu.SemaphoreType.DMA,
            pltpu.SemaphoreType.DMA,
            pltpu.SemaphoreType.DMA,
            pltpu.SemaphoreType.DMA,
        ],
    )
    return kern


def _make_lookup(hist: int, num_workers: int, steps: int):
    n_btiles = steps * num_workers // hist
    mesh = plsc.VectorSubcoreMesh(core_axis_name="c", subcore_axis_name="s")

    def body(idx_hbm, table_hbm, out_hbm, idx_v, g0, g1, t0, t1,
             sg0, sg1, st0, st1):
        nc = mesh.num_cores
        wid = lax.axis_index("s") * nc + lax.axis_index("c")
        beta0 = wid * steps
        gb = (g0, g1)
        tb = (t0, t1)
        sg = (sg0, sg1)
        st = (st0, st1)

        pltpu.sync_copy(idx_hbm.at[wid], idx_v)

        lane_iota = lax.iota(jnp.int32, LANES)
        scat_dg = [(lane_iota + q * LANES) // 8 for q in range(DIM // LANES)]
        scat_di = [(lane_iota + q * LANES) % 8 for q in range(DIM // LANES)]

        def start_gather(j, b):
            pltpu.async_copy(table_hbm.at[idx_v.at[j]], gb[b], sg[b])

        def transpose_scale(b):
            src = gb[b]
            dst = tb[b]

            @pl.loop(0, CHUNK, unroll=4)
            def _(t):
                t_vec = lane_iota * 0 + t
                for q in range(DIM // LANES):
                    vals = src[t, pl.ds(q * LANES, LANES)]
                    plsc.store_scatter(
                        dst, [scat_dg[q], scat_di[q], t_vec], vals)

        def drain_out(b):
            pltpu.make_async_copy(
                tb[b].at[:, :, pl.ds(0, CHUNK)], out_hbm.at[0, :, 0],
                st[b]).wait()

        def pipe_step(j, b, *, out_wait, prefetch):
            beta = beta0 + j
            h = beta // n_btiles
            bt = beta - h * n_btiles
            pltpu.make_async_copy(table_hbm.at[idx_v.at[j]], gb[b], sg[b]).wait()
            if out_wait:
                drain_out(b)
            transpose_scale(b)
            if prefetch:
                start_gather(j + 2, b)
            pltpu.async_copy(
                tb[b].at[:, :, pl.ds(0, CHUNK)], out_hbm.at[h, :, bt], st[b])

        start_gather(0, 0)
        start_gather(1, 1)
        pipe_step(0, 0, out_wait=False, prefetch=True)
        pipe_step(1, 1, out_wait=False, prefetch=True)

        @pl.loop(1, steps // 2 - 1)
        def _(g):
            pipe_step(2 * g, 0, out_wait=True, prefetch=True)
            pipe_step(2 * g + 1, 1, out_wait=True, prefetch=True)

        pipe_step(steps - 2, 0, out_wait=True, prefetch=False)
        pipe_step(steps - 1, 1, out_wait=True, prefetch=False)
        drain_out(0)
        drain_out(1)

    kern = pl.kernel(
        body,
        out_type=jax.ShapeDtypeStruct(
            (hist, DIM // 8, n_btiles, 8, CHUNK), jnp.float32),
        mesh=mesh,
        compiler_params=pltpu.CompilerParams(
            use_tc_tiling_on_sc=False, needs_layout_passes=False),
        scratch_types=[
            pltpu.VMEM((steps, CHUNK), jnp.int32),
            pltpu.VMEM((CHUNK, DIM), jnp.float32),
            pltpu.VMEM((CHUNK, DIM), jnp.float32),
            pltpu.VMEM((DIM // 8, 8, PADC), jnp.float32),
            pltpu.VMEM((DIM // 8, 8, PADC), jnp.float32),
            pltpu.SemaphoreType.DMA,
            pltpu.SemaphoreType.DMA,
            pltpu.SemaphoreType.DMA,
            pltpu.SemaphoreType.DMA,
        ],
    )
    return kern


def kernel(x, w_embed):
    batch, hist = x.shape
    total = batch * hist
    info = plsc.get_sparse_core_info()
    num_workers = info.num_cores * info.num_subcores
    steps = total // (num_workers * CHUNK)
    assert steps * num_workers * CHUNK == total
    assert batch % CHUNK == 0
    t64 = (w_embed[VOCAB - TAIL:, :] * SCALE).reshape(TAIL // 2, 2 * DIM)
    w_lin = _make_prep(num_workers)(w_embed.T, t64)
    table = w_lin.reshape(VOCAB, DIM)
    idx = x.T.reshape(num_workers, steps, CHUNK).astype(jnp.int32)
    out5 = _make_lookup(hist, num_workers, steps)(idx, table)
    out = out5.transpose(2, 4, 0, 1, 3).reshape(batch, hist, DIM)
    return out


# speedup vs baseline: 1.8376x; 1.8376x over previous
import math

import jax
import jax.numpy as jnp
from jax import lax
from jax.experimental import pallas as pl
from jax.experimental.pallas import tpu as pltpu
from jax.experimental.pallas import tpu_sc as plsc

DIM = 64
SCALE = math.sqrt(DIM)
LANES = 16
CHUNK = 128
PADC = CHUNK + 5


def _make_kernel(hist: int, num_workers: int, steps: int):
    n_btiles = steps * num_workers // hist
    mesh = plsc.VectorSubcoreMesh(core_axis_name="c", subcore_axis_name="s")

    def body(idx_hbm, table_hbm, out_hbm, idx_v, g0, g1, t0, t1,
             sg0, sg1, st0, st1):
        nc = mesh.num_cores
        wid = lax.axis_index("s") * nc + lax.axis_index("c")
        beta0 = wid * steps
        gb = (g0, g1)
        tb = (t0, t1)
        sg = (sg0, sg1)
        st = (st0, st1)

        pltpu.sync_copy(idx_hbm.at[wid], idx_v)

        lane_iota = lax.iota(jnp.int32, LANES)
        scat_dg = [(lane_iota + q * LANES) // 8 for q in range(DIM // LANES)]
        scat_di = [(lane_iota + q * LANES) % 8 for q in range(DIM // LANES)]

        def start_gather(j, b):
            pltpu.async_copy(table_hbm.at[idx_v.at[j]], gb[b], sg[b])

        def transpose_scale(b):
            src = gb[b]
            dst = tb[b]

            @pl.loop(0, CHUNK, unroll=4)
            def _(t):
                t_vec = lane_iota * 0 + t
                for q in range(DIM // LANES):
                    vals = src[t, pl.ds(q * LANES, LANES)]
                    plsc.store_scatter(
                        dst, [scat_dg[q], scat_di[q], t_vec], vals * SCALE)

        def drain_out(b):
            pltpu.make_async_copy(
                tb[b].at[:, :, pl.ds(0, CHUNK)], out_hbm.at[0, :, 0],
                st[b]).wait()

        def pipe_step(j, b, *, out_wait, prefetch):
            beta = beta0 + j
            h = beta // n_btiles
            bt = beta - h * n_btiles
            pltpu.make_async_copy(table_hbm.at[idx_v.at[j]], gb[b], sg[b]).wait()
            if out_wait:
                drain_out(b)
            transpose_scale(b)
            if prefetch:
                start_gather(j + 2, b)
            pltpu.async_copy(
                tb[b].at[:, :, pl.ds(0, CHUNK)], out_hbm.at[h, :, bt], st[b])

        start_gather(0, 0)
        start_gather(1, 1)
        pipe_step(0, 0, out_wait=False, prefetch=True)
        pipe_step(1, 1, out_wait=False, prefetch=True)

        @pl.loop(1, steps // 2 - 1)
        def _(g):
            pipe_step(2 * g, 0, out_wait=True, prefetch=True)
            pipe_step(2 * g + 1, 1, out_wait=True, prefetch=True)

        pipe_step(steps - 2, 0, out_wait=True, prefetch=False)
        pipe_step(steps - 1, 1, out_wait=True, prefetch=False)
        drain_out(0)
        drain_out(1)

    kern = pl.kernel(
        body,
        out_type=jax.ShapeDtypeStruct(
            (hist, DIM // 8, n_btiles, 8, CHUNK), jnp.float32),
        mesh=mesh,
        compiler_params=pltpu.CompilerParams(
            use_tc_tiling_on_sc=False, needs_layout_passes=False),
        scratch_types=[
            pltpu.VMEM((steps, CHUNK), jnp.int32),
            pltpu.VMEM((CHUNK, DIM), jnp.float32),
            pltpu.VMEM((CHUNK, DIM), jnp.float32),
            pltpu.VMEM((DIM // 8, 8, PADC), jnp.float32),
            pltpu.VMEM((DIM // 8, 8, PADC), jnp.float32),
            pltpu.SemaphoreType.DMA,
            pltpu.SemaphoreType.DMA,
            pltpu.SemaphoreType.DMA,
            pltpu.SemaphoreType.DMA,
        ],
    )
    return kern


def kernel(x, w_embed):
    batch, hist = x.shape
    total = batch * hist
    info = plsc.get_sparse_core_info()
    num_workers = info.num_cores * info.num_subcores
    steps = total // (num_workers * CHUNK)
    assert steps * num_workers * CHUNK == total
    assert batch % CHUNK == 0
    idx = x.T.reshape(num_workers, steps, CHUNK).astype(jnp.int32)
    out5 = _make_kernel(hist, num_workers, steps)(idx, w_embed)
    out = out5.transpose(2, 4, 0, 1, 3).reshape(batch, hist, DIM)
    return out
